# baseline (speedup 1.0000x reference)
import jax
import jax.numpy as jnp
from jax import lax
from jax.experimental import pallas as pl
from jax.experimental.pallas import tpu as pltpu
from jax.experimental.pallas import tpu_sc as plsc

_K = 100000
_D = 128
_NC = 2
_NS = 16
_NW = _NC * _NS
_CH = 136
_Q = 8
_S = _NW * _CH * _Q
_T = _K - _S
_GRP = (_CH + 15) // 16
_NBUF = 4

_mesh = plsc.VectorSubcoreMesh(core_axis_name="c", subcore_axis_name="s")
_params = pltpu.CompilerParams(needs_layout_passes=False)


_TB = 5000
_TSTEPS = (_T + _TB - 1) // _TB


def _tc_body(x_ref, p_ref, val_out, idx_out, best_val, best_idx):
    i = pl.program_id(0)

    @pl.when(i == 0)
    def _init():
        best_val[0] = -jnp.inf
        best_idx[0] = 0

    p = p_ref[...]
    x = x_ref[...]
    dots = jax.lax.dot_general(
        p, x, (((1,), (1,)), ((), ())),
        preferred_element_type=jnp.float32,
        precision=jax.lax.Precision.HIGHEST,
    )[:, 0]
    n2 = jnp.sum(p * p, axis=1)
    t = dots * jnp.abs(dots) / jnp.maximum(n2, 1e-16)
    rows = i * _TB + jax.lax.broadcasted_iota(jnp.int32, (_TB,), 0)
    t = jnp.where(rows < _T, t, -jnp.inf)

    local_max = jnp.max(t)

    @pl.when(local_max > best_val[0])
    def _upd():
        best_val[0] = local_max
        best_idx[0] = i * _TB + jnp.argmax(t)

    @pl.when(i == pl.num_programs(0) - 1)
    def _fin():
        val_out[...] = jnp.full((1, _D), best_val[0], jnp.float32)
        idx_out[...] = jnp.full((1, _D), best_idx[0], jnp.int32)


def _tc_scan(x, patterns):
    return pl.pallas_call(
        _tc_body,
        grid=(_TSTEPS,),
        in_specs=[
            pl.BlockSpec((1, _D), lambda i: (0, 0)),
            pl.BlockSpec((_TB, _D), lambda i: (i, 0)),
        ],
        out_specs=[
            pl.BlockSpec((1, _D), lambda i: (0, 0)),
            pl.BlockSpec((1, _D), lambda i: (0, 0)),
        ],
        out_shape=[
            jax.ShapeDtypeStruct((1, _D), jnp.float32),
            jax.ShapeDtypeStruct((1, _D), jnp.int32),
        ],
        scratch_shapes=[
            pltpu.SMEM((1,), jnp.float32),
            pltpu.SMEM((1,), jnp.int32),
        ],
        compiler_params=pltpu.CompilerParams(
            dimension_semantics=("arbitrary",),
        ),
    )(x.reshape(1, _D), patterns)


def _score_chunk(bref, xv, row0, carry):
    lane = lax.iota(jnp.int32, 16)
    lrow_cs = [jnp.minimum(g * 16 + lane, _CH - 1) for g in range(_GRP)]
    zeros = jnp.zeros((16,), jnp.float32)

    def jstep(j, accs):
        xvec = xv[pl.ds(j, 16)]
        col = lane + j
        col = jnp.where(col >= _D, col - _D, col)
        new = []
        for g in range(_GRP):
            v = plsc.load_gather(bref, [lrow_cs[g], col])
            new.append(accs[2 * g] + v * xvec)
            new.append(accs[2 * g + 1] + v * v)
        return tuple(new)

    accs = lax.fori_loop(0, _D, jstep, (zeros,) * (2 * _GRP))

    bv, bi = carry
    for g in range(_GRP):
        dot, n2 = accs[2 * g], accs[2 * g + 1]
        t = dot * jnp.abs(dot) / jnp.maximum(n2, 1e-16)
        lrow = g * 16 + lane
        t = jnp.where(lrow < _CH, t, -jnp.inf)
        upd = t > bv
        bv = jnp.where(upd, t, bv)
        bi = jnp.where(upd, row0 + lrow, bi)
    return bv, bi


def _chunk_start(w, c):
    return pl.multiple_of(_T + (w + _NW * c) * _CH, 8)


def _scan_body(x_hbm, p_hbm, val_out, idx_out, xv, buf, valv, idxv, *sems):
    wid = lax.axis_index("s") * _NC + lax.axis_index("c")

    pltpu.sync_copy(x_hbm, xv.at[pl.ds(0, _D)])
    xv[pl.ds(_D, 16)] = xv[pl.ds(0, 16)]

    def start(c, s):
        pltpu.make_async_copy(
            p_hbm.at[pl.ds(_chunk_start(wid, c), _CH)], buf.at[s],
            sems[s]).start()

    def wait(c, s):
        pltpu.make_async_copy(
            p_hbm.at[pl.ds(_chunk_start(wid, c), _CH)], buf.at[s],
            sems[s]).wait()

    for s in range(min(_NBUF, _Q)):
        start(s, s)

    carry = (jnp.full((16,), -jnp.inf, jnp.float32),
             jnp.zeros((16,), jnp.int32))

    def ring(k, c):
        for s in range(_NBUF):
            ch = _NBUF * k + s
            wait(ch, s)
            c = _score_chunk(buf.at[s], xv, _chunk_start(wid, ch), c)

            @pl.when(ch + _NBUF < _Q)
            def _():
                start(ch + _NBUF, s)
        return c

    carry = lax.fori_loop(0, _Q // _NBUF, ring, carry)
    for ch in range((_Q // _NBUF) * _NBUF, _Q):
        s = ch % _NBUF
        wait(ch, s)
        carry = _score_chunk(buf.at[s], xv, _chunk_start(wid, ch), carry)

    bv, bi = carry
    valv[...] = bv
    idxv[...] = bi
    off = pl.multiple_of(wid * 16, 8)
    pltpu.sync_copy(valv, val_out.at[pl.ds(off, 16)])
    pltpu.sync_copy(idxv, idx_out.at[pl.ds(off, 16)])


_sc_scan = pl.kernel(
    _scan_body,
    out_type=[
        jax.ShapeDtypeStruct((_NW * 16,), jnp.float32),
        jax.ShapeDtypeStruct((_NW * 16,), jnp.int32),
    ],
    mesh=_mesh,
    scratch_types=[
        pltpu.VMEM((_D + 16,), jnp.float32),
        pltpu.VMEM((_NBUF, _CH, _D), jnp.float32),
        pltpu.VMEM((16,), jnp.float32),
        pltpu.VMEM((16,), jnp.int32),
    ] + [pltpu.SemaphoreType.DMA] * _NBUF,
    compiler_params=_params,
)


def _merge_body(p_ref, val_ref, idx_ref, tcv_ref, tci_ref, out_ref,
                row_v, sem):
    v = val_ref[...]
    iv = idx_ref[...]
    tv = tcv_ref[...]
    ti = tci_ref[...]

    gmax = jnp.maximum(jnp.max(v), jnp.max(tv))
    big = jnp.int32(2**31 - 1)
    gidx = jnp.minimum(
        jnp.min(jnp.where(v == gmax, iv, big)),
        jnp.min(jnp.where(tv == gmax, ti, big)))

    pltpu.make_async_copy(p_ref.at[pl.ds(gidx, 1)], row_v, sem).start()
    pltpu.make_async_copy(p_ref.at[pl.ds(gidx, 1)], row_v, sem).wait()
    out_ref[...] = row_v[...]


def _merge(patterns, sc_vals, sc_idxs, tc_val, tc_idx):
    return pl.pallas_call(
        _merge_body,
        in_specs=[
            pl.BlockSpec(memory_space=pl.ANY),
            pl.BlockSpec((_NW // 8, _D), lambda: (0, 0)),
            pl.BlockSpec((_NW // 8, _D), lambda: (0, 0)),
            pl.BlockSpec((1, _D), lambda: (0, 0)),
            pl.BlockSpec((1, _D), lambda: (0, 0)),
        ],
        out_specs=pl.BlockSpec((1, _D), lambda: (0, 0)),
        out_shape=jax.ShapeDtypeStruct((1, _D), jnp.float32),
        scratch_shapes=[
            pltpu.VMEM((1, _D), jnp.float32),
            pltpu.SemaphoreType.DMA,
        ],
    )(patterns, sc_vals.reshape(_NW // 8, _D), sc_idxs.reshape(_NW // 8, _D),
      tc_val, tc_idx)


def kernel(x, patterns):
    sc_vals, sc_idxs = _sc_scan(x, patterns)
    tc_val, tc_idx = _tc_scan(x, patterns)
    return _merge(patterns, sc_vals, sc_idxs, tc_val, tc_idx)[0]


# speedup vs baseline: 1.0968x; 1.0968x over previous
import jax
import jax.numpy as jnp
from jax import lax
from jax.experimental import pallas as pl
from jax.experimental.pallas import tpu as pltpu
from jax.experimental.pallas import tpu_sc as plsc

_K = 100000
_D = 128
_NC = 2
_NS = 16
_NW = _NC * _NS
_CH = 136
_Q = 8
_S = _NW * _CH * _Q
_T = _K - _S
_GRP = (_CH + 15) // 16
_NBUF = 4

_mesh = plsc.VectorSubcoreMesh(core_axis_name="c", subcore_axis_name="s")
_params = pltpu.CompilerParams(needs_layout_passes=False)


_TNB = 4
_TCB = 4080
_TNCH = (_T + _TCB - 1) // _TCB


def _tc_body(x_ref, p_hbm, val_out, idx_out, buf, *sems):
    x = x_ref[...]

    def start(c, s):
        pltpu.make_async_copy(
            p_hbm.at[pl.ds(pl.multiple_of(c * _TCB, 8), _TCB)], buf.at[s],
            sems[s]).start()

    def wait(c, s):
        pltpu.make_async_copy(
            p_hbm.at[pl.ds(pl.multiple_of(c * _TCB, 8), _TCB)], buf.at[s],
            sems[s]).wait()

    for s in range(_TNB):
        start(s, s)

    def score(c, s, carry):
        bv, bi = carry
        wait(c, s)
        p = buf[s]

        @pl.when(c + _TNB < _TNCH)
        def _():
            start(c + _TNB, s)

        dots = jax.lax.dot_general(
            p, x, (((1,), (1,)), ((), ())),
            preferred_element_type=jnp.float32,
            precision=jax.lax.Precision.HIGHEST,
        )[:, 0]
        n2 = jnp.sum(p * p, axis=1)
        t = dots * jnp.abs(dots) / jnp.maximum(n2, 1e-16)
        rows = c * _TCB + jax.lax.broadcasted_iota(jnp.int32, (_TCB,), 0)
        t = jnp.where(rows < _T, t, -jnp.inf)

        m = jnp.max(t)
        better = m > bv
        bi = jnp.where(better, c * _TCB + jnp.argmax(t), bi)
        bv = jnp.where(better, m, bv)
        return bv, bi

    def ring(k, carry):
        for s in range(_TNB):
            carry = score(_TNB * k + s, s, carry)
        return carry

    carry = (-jnp.inf, jnp.int32(0))
    carry = lax.fori_loop(0, _TNCH // _TNB, ring, carry)
    for c in range((_TNCH // _TNB) * _TNB, _TNCH):
        carry = score(c, c % _TNB, carry)

    bv, bi = carry
    val_out[...] = jnp.full((1, _D), bv, jnp.float32)
    idx_out[...] = jnp.full((1, _D), bi, jnp.int32)


def _tc_scan(x, patterns):
    return pl.pallas_call(
        _tc_body,
        in_specs=[
            pl.BlockSpec((1, _D), lambda: (0, 0)),
            pl.BlockSpec(memory_space=pl.ANY),
        ],
        out_specs=[
            pl.BlockSpec((1, _D), lambda: (0, 0)),
            pl.BlockSpec((1, _D), lambda: (0, 0)),
        ],
        out_shape=[
            jax.ShapeDtypeStruct((1, _D), jnp.float32),
            jax.ShapeDtypeStruct((1, _D), jnp.int32),
        ],
        scratch_shapes=[
            pltpu.VMEM((_TNB, _TCB, _D), jnp.float32),
        ] + [pltpu.SemaphoreType.DMA] * _TNB,
    )(x.reshape(1, _D), patterns)


def _score_chunk(bref, xv, row0, carry):
    lane = lax.iota(jnp.int32, 16)
    lrow_cs = [jnp.minimum(g * 16 + lane, _CH - 1) for g in range(_GRP)]
    zeros = jnp.zeros((16,), jnp.float32)

    def jstep(j, accs):
        xvec = xv[pl.ds(j, 16)]
        col = lane + j
        col = jnp.where(col >= _D, col - _D, col)
        new = []
        for g in range(_GRP):
            v = plsc.load_gather(bref, [lrow_cs[g], col])
            new.append(accs[2 * g] + v * xvec)
            new.append(accs[2 * g + 1] + v * v)
        return tuple(new)

    accs = lax.fori_loop(0, _D, jstep, (zeros,) * (2 * _GRP))

    bv, bi = carry
    for g in range(_GRP):
        dot, n2 = accs[2 * g], accs[2 * g + 1]
        t = dot * jnp.abs(dot) / jnp.maximum(n2, 1e-16)
        lrow = g * 16 + lane
        t = jnp.where(lrow < _CH, t, -jnp.inf)
        upd = t > bv
        bv = jnp.where(upd, t, bv)
        bi = jnp.where(upd, row0 + lrow, bi)
    return bv, bi


def _chunk_start(w, c):
    return pl.multiple_of(_T + (w + _NW * c) * _CH, 8)


def _scan_body(x_hbm, p_hbm, val_out, idx_out, xv, buf, valv, idxv, *sems):
    wid = lax.axis_index("s") * _NC + lax.axis_index("c")

    pltpu.sync_copy(x_hbm, xv.at[pl.ds(0, _D)])
    xv[pl.ds(_D, 16)] = xv[pl.ds(0, 16)]

    def start(c, s):
        pltpu.make_async_copy(
            p_hbm.at[pl.ds(_chunk_start(wid, c), _CH)], buf.at[s],
            sems[s]).start()

    def wait(c, s):
        pltpu.make_async_copy(
            p_hbm.at[pl.ds(_chunk_start(wid, c), _CH)], buf.at[s],
            sems[s]).wait()

    for s in range(min(_NBUF, _Q)):
        start(s, s)

    carry = (jnp.full((16,), -jnp.inf, jnp.float32),
             jnp.zeros((16,), jnp.int32))

    def ring(k, c):
        for s in range(_NBUF):
            ch = _NBUF * k + s
            wait(ch, s)
            c = _score_chunk(buf.at[s], xv, _chunk_start(wid, ch), c)

            @pl.when(ch + _NBUF < _Q)
            def _():
                start(ch + _NBUF, s)
        return c

    carry = lax.fori_loop(0, _Q // _NBUF, ring, carry)
    for ch in range((_Q // _NBUF) * _NBUF, _Q):
        s = ch % _NBUF
        wait(ch, s)
        carry = _score_chunk(buf.at[s], xv, _chunk_start(wid, ch), carry)

    bv, bi = carry
    valv[...] = bv
    idxv[...] = bi
    off = pl.multiple_of(wid * 16, 8)
    pltpu.sync_copy(valv, val_out.at[pl.ds(off, 16)])
    pltpu.sync_copy(idxv, idx_out.at[pl.ds(off, 16)])


_sc_scan = pl.kernel(
    _scan_body,
    out_type=[
        jax.ShapeDtypeStruct((_NW * 16,), jnp.float32),
        jax.ShapeDtypeStruct((_NW * 16,), jnp.int32),
    ],
    mesh=_mesh,
    scratch_types=[
        pltpu.VMEM((_D + 16,), jnp.float32),
        pltpu.VMEM((_NBUF, _CH, _D), jnp.float32),
        pltpu.VMEM((16,), jnp.float32),
        pltpu.VMEM((16,), jnp.int32),
    ] + [pltpu.SemaphoreType.DMA] * _NBUF,
    compiler_params=_params,
)


def _merge_body(p_ref, val_ref, idx_ref, tcv_ref, tci_ref, out_ref,
                row_v, sem):
    v = val_ref[...]
    iv = idx_ref[...]
    tv = tcv_ref[...]
    ti = tci_ref[...]

    gmax = jnp.maximum(jnp.max(v), jnp.max(tv))
    big = jnp.int32(2**31 - 1)
    gidx = jnp.minimum(
        jnp.min(jnp.where(v == gmax, iv, big)),
        jnp.min(jnp.where(tv == gmax, ti, big)))

    pltpu.make_async_copy(p_ref.at[pl.ds(gidx, 1)], row_v, sem).start()
    pltpu.make_async_copy(p_ref.at[pl.ds(gidx, 1)], row_v, sem).wait()
    out_ref[...] = row_v[...]


def _merge(patterns, sc_vals, sc_idxs, tc_val, tc_idx):
    return pl.pallas_call(
        _merge_body,
        in_specs=[
            pl.BlockSpec(memory_space=pl.ANY),
            pl.BlockSpec((_NW // 8, _D), lambda: (0, 0)),
            pl.BlockSpec((_NW // 8, _D), lambda: (0, 0)),
            pl.BlockSpec((1, _D), lambda: (0, 0)),
            pl.BlockSpec((1, _D), lambda: (0, 0)),
        ],
        out_specs=pl.BlockSpec((1, _D), lambda: (0, 0)),
        out_shape=jax.ShapeDtypeStruct((1, _D), jnp.float32),
        scratch_shapes=[
            pltpu.VMEM((1, _D), jnp.float32),
            pltpu.SemaphoreType.DMA,
        ],
    )(patterns, sc_vals.reshape(_NW // 8, _D), sc_idxs.reshape(_NW // 8, _D),
      tc_val, tc_idx)


def kernel(x, patterns):
    sc_vals, sc_idxs = _sc_scan(x, patterns)
    tc_val, tc_idx = _tc_scan(x, patterns)
    return _merge(patterns, sc_vals, sc_idxs, tc_val, tc_idx)[0]
